# Initial kernel scaffold; baseline (speedup 1.0000x reference)
import jax
import jax.numpy as jnp
from jax.experimental import pallas as pl


def kernel(hidden_states, gate_w, w1, w3, w2):
    raise NotImplementedError("write your pallas kernel here")


# speedup vs baseline: 1.7313x; 1.7313x over previous
import functools

import jax
import jax.numpy as jnp
from jax import lax
from jax.experimental import pallas as pl
from jax.experimental.pallas import tpu as pltpu
from jax.experimental.pallas import tpu_sc as plsc

T, H, F, E, K = 4096, 1024, 2048, 8, 2
BLK = 256
P = T * K + E * BLK
NB = P // BLK
CHUNK = 512
NCH = T // CHUNK

NC_SC, NS_SC = 2, 16
NW = NC_SC * NS_SC
TPW = T // NW
SC_CH = 32
NCH_SC = TPW // SC_CH


def _router_meta_body(hs_ref, gw_ref, logits_ref, w0_ref, w1_ref, slots_ref,
                      ends_ref, carry, eids):
    i = pl.program_id(0)

    @pl.when(i < NCH)
    def _chunk():
        @pl.when(i == 0)
        def _init():
            carry[...] = jnp.zeros_like(carry)

        x = hs_ref[...]
        gw = gw_ref[...]
        logits = lax.dot_general(x, gw, (((1,), (1,)), ((), ())),
                                 preferred_element_type=jnp.float32)
        logits_ref[pl.ds(i * CHUNK, CHUNK), :] = logits

        lane = lax.broadcasted_iota(jnp.int32, (CHUNK, E), 1)
        m0 = jnp.max(logits, axis=1, keepdims=True)
        idx0 = jnp.min(jnp.where(logits == m0, lane, E), axis=1, keepdims=True)
        sel0 = lane == idx0
        l2 = jnp.where(sel0, -jnp.inf, logits)
        m1 = jnp.max(l2, axis=1, keepdims=True)
        idx1 = jnp.min(jnp.where(l2 == m1, lane, E), axis=1, keepdims=True)
        sel1 = lane == idx1

        p = jnp.exp(logits - m0)
        p0 = jnp.sum(jnp.where(sel0, p, 0.0), axis=1, keepdims=True)
        p1 = jnp.sum(jnp.where(sel1, p, 0.0), axis=1, keepdims=True)
        w0 = p0 / (p0 + p1)
        w1 = p1 / (p0 + p1)
        w0_ref[pl.ds(i * CHUNK, CHUNK), :] = jnp.broadcast_to(w0, (CHUNK, 16))
        w1_ref[pl.ds(i * CHUNK, CHUNK), :] = jnp.broadcast_to(w1, (CHUNK, 16))
        eids[pl.ds(i * CHUNK, CHUNK), :] = (
            jnp.where(lane == 0, idx0, 0) + jnp.where(lane == 1, idx1, 0))

        oh0 = sel0.astype(jnp.bfloat16)
        oh1 = sel1.astype(jnp.bfloat16)
        ohs = oh0 + oh1
        row = lax.broadcasted_iota(jnp.int32, (CHUNK, CHUNK), 0)
        col = lax.broadcasted_iota(jnp.int32, (CHUNK, CHUNK), 1)
        tril = (col < row).astype(jnp.bfloat16)
        cum = lax.dot_general(tril, ohs, (((1,), (0,)), ((), ())),
                              preferred_element_type=jnp.float32)
        base = cum + carry[0:1, 0:E]
        r0 = jnp.sum(jnp.where(sel0, base, 0.0), axis=1, keepdims=True)
        r1 = jnp.sum(jnp.where(sel1, base, 0.0), axis=1, keepdims=True)
        slots_ref[pl.ds(i * CHUNK, CHUNK), :] = (
            jnp.where(lane == 0, r0, 0.0)
            + jnp.where(lane == 1, r1, 0.0)).astype(jnp.int32)
        carry[0:1, 0:E] = carry[0:1, 0:E] + jnp.sum(
            ohs.astype(jnp.float32), axis=0, keepdims=True)

    @pl.when(i == NCH)
    def _final():
        counts = carry[0:1, 0:E].astype(jnp.int32)
        padded = ((counts + (BLK - 1)) // BLK) * BLK
        lt = (lax.broadcasted_iota(jnp.int32, (E, E), 0)
              < lax.broadcasted_iota(jnp.int32, (E, E), 1)).astype(jnp.bfloat16)
        offs = lax.dot_general(padded.astype(jnp.bfloat16), lt,
                               (((1,), (0,)), ((), ())),
                               preferred_element_type=jnp.float32)
        offs_i = offs.astype(jnp.int32)
        ends_ref[0:1, 0:E] = offs_i + padded

        lane = lax.broadcasted_iota(jnp.int32, (T, E), 1)
        eall = eids[...]
        e0 = eall[:, 0:1]
        e1 = eall[:, 1:2]
        off0 = jnp.sum(jnp.where(e0 == lane, offs_i, 0), axis=1, keepdims=True)
        off1 = jnp.sum(jnp.where(e1 == lane, offs_i, 0), axis=1, keepdims=True)
        slots_ref[...] = (slots_ref[...]
                          + jnp.where(lane == 0, off0, 0)
                          + jnp.where(lane == 1, off1, 0))


def _router_meta(hs2, gate_w):
    return pl.pallas_call(
        _router_meta_body,
        grid=(NCH + 1,),
        in_specs=[
            pl.BlockSpec((CHUNK, H), lambda i: (jnp.minimum(i, NCH - 1), 0)),
            pl.BlockSpec((E, H), lambda i: (0, 0)),
        ],
        out_specs=[
            pl.BlockSpec((T, E), lambda i: (0, 0)),
            pl.BlockSpec((T, 16), lambda i: (0, 0)),
            pl.BlockSpec((T, 16), lambda i: (0, 0)),
            pl.BlockSpec((T, E), lambda i: (0, 0)),
            pl.BlockSpec((8, 128), lambda i: (0, 0)),
        ],
        out_shape=[
            jax.ShapeDtypeStruct((T, E), jnp.float32),
            jax.ShapeDtypeStruct((T, 16), jnp.float32),
            jax.ShapeDtypeStruct((T, 16), jnp.float32),
            jax.ShapeDtypeStruct((T, E), jnp.int32),
            jax.ShapeDtypeStruct((8, 128), jnp.int32),
        ],
        scratch_shapes=[
            pltpu.VMEM((8, 128), jnp.float32),
            pltpu.VMEM((T, E), jnp.int32),
        ],
    )(hs2, gate_w)


@functools.cache
def _sc_kernels():
    mesh = plsc.VectorSubcoreMesh(core_axis_name="c", subcore_axis_name="s")

    @functools.partial(
        pl.kernel,
        out_type=jax.ShapeDtypeStruct((P, H), jnp.float32),
        mesh=mesh,
        scratch_types=[
            pltpu.VMEM((SC_CH, H), jnp.float32),
            pltpu.VMEM((SC_CH,), jnp.int32),
            pltpu.VMEM((SC_CH,), jnp.int32),
            pltpu.SemaphoreType.DMA,
            pltpu.SemaphoreType.DMA,
        ],
    )
    def sc_scatter(hs_hbm, slot0_hbm, slot1_hbm, x_hbm, xbuf, idx0, idx1,
                   sem0, sem1):
        wid = lax.axis_index("s") * NC_SC + lax.axis_index("c")

        def body(c, _):
            base = wid * TPW + c * SC_CH
            pltpu.sync_copy(hs_hbm.at[pl.ds(base, SC_CH)], xbuf)
            pltpu.sync_copy(slot0_hbm.at[wid, c], idx0)
            pltpu.sync_copy(slot1_hbm.at[wid, c], idx1)
            d0 = pltpu.async_copy(xbuf, x_hbm.at[idx0], sem0)
            d1 = pltpu.async_copy(xbuf, x_hbm.at[idx1], sem1)
            d0.wait()
            d1.wait()
            return ()

        lax.fori_loop(0, NCH_SC, body, ())

    @functools.partial(
        pl.kernel,
        out_type=jax.ShapeDtypeStruct((T, H), jnp.float32),
        mesh=mesh,
        scratch_types=[
            pltpu.VMEM((SC_CH, H), jnp.float32),
            pltpu.VMEM((SC_CH, H), jnp.float32),
            pltpu.VMEM((SC_CH, H), jnp.float32),
            pltpu.VMEM((SC_CH,), jnp.int32),
            pltpu.VMEM((SC_CH,), jnp.int32),
            pltpu.VMEM((SC_CH * 16,), jnp.float32),
            pltpu.VMEM((SC_CH * 16,), jnp.float32),
            pltpu.SemaphoreType.DMA,
            pltpu.SemaphoreType.DMA,
        ],
    )
    def sc_combine(y_hbm, slot0_hbm, slot1_hbm, w0_hbm, w1_hbm, out_hbm,
                   y0, y1, ob, idx0, idx1, wv0, wv1, sem0, sem1):
        wid = lax.axis_index("s") * NC_SC + lax.axis_index("c")

        def body(c, _):
            base = wid * TPW + c * SC_CH
            pltpu.sync_copy(slot0_hbm.at[wid, c], idx0)
            pltpu.sync_copy(slot1_hbm.at[wid, c], idx1)
            pltpu.sync_copy(w0_hbm.at[wid, c], wv0)
            pltpu.sync_copy(w1_hbm.at[wid, c], wv1)
            d0 = pltpu.async_copy(y_hbm.at[idx0], y0, sem0)
            d1 = pltpu.async_copy(y_hbm.at[idx1], y1, sem1)
            d0.wait()
            d1.wait()

            def rows(r, _):
                woff = pl.multiple_of(r * 16, 16)
                wb0 = wv0[pl.ds(woff, 16)]
                wb1 = wv1[pl.ds(woff, 16)]

                def cols(v, _):
                    for u in range(8):
                        off = pl.multiple_of(v * 128 + u * 16, 16)
                        ob[r, pl.ds(off, 16)] = (
                            wb0 * y0[r, pl.ds(off, 16)]
                            + wb1 * y1[r, pl.ds(off, 16)])
                    return ()

                lax.fori_loop(0, H // 128, cols, ())
                return ()

            lax.fori_loop(0, SC_CH, rows, ())
            pltpu.sync_copy(ob, out_hbm.at[pl.ds(base, SC_CH)])
            return ()

        lax.fori_loop(0, NCH_SC, body, ())

    return sc_scatter, sc_combine


def _gemm_body(ends_ref, x_ref, w1_ref, w3_ref, w2_ref, y_ref):
    x = x_ref[...].astype(jnp.bfloat16)
    a = lax.dot_general(x, w1_ref[0], (((1,), (1,)), ((), ())),
                        preferred_element_type=jnp.float32)
    b = lax.dot_general(x, w3_ref[0], (((1,), (1,)), ((), ())),
                        preferred_element_type=jnp.float32)
    g = a * (1.0 / (1.0 + jnp.exp(-a))) * b
    y_ref[...] = lax.dot_general(g.astype(jnp.bfloat16), w2_ref[0],
                                 (((1,), (1,)), ((), ())),
                                 preferred_element_type=jnp.float32)


def _block_expert(i, ends_ref):
    s = jnp.int32(0)
    for e in range(E):
        s = s + jnp.where(i * BLK >= ends_ref[e], 1, 0).astype(jnp.int32)
    return jnp.minimum(s, E - 1)


def _grouped_gemm(ends, x_sorted, w1b, w3b, w2b):
    grid_spec = pltpu.PrefetchScalarGridSpec(
        num_scalar_prefetch=1,
        grid=(NB,),
        in_specs=[
            pl.BlockSpec((BLK, H), lambda i, er: (i, 0)),
            pl.BlockSpec((1, F, H), lambda i, er: (_block_expert(i, er), 0, 0)),
            pl.BlockSpec((1, F, H), lambda i, er: (_block_expert(i, er), 0, 0)),
            pl.BlockSpec((1, H, F), lambda i, er: (_block_expert(i, er), 0, 0)),
        ],
        out_specs=pl.BlockSpec((BLK, H), lambda i, er: (i, 0)),
    )
    return pl.pallas_call(
        _gemm_body,
        grid_spec=grid_spec,
        out_shape=jax.ShapeDtypeStruct((P, H), jnp.float32),
    )(ends, x_sorted, w1b, w3b, w2b)


def kernel(hidden_states, gate_w, w1, w3, w2):
    b, s, h = hidden_states.shape
    hs2 = hidden_states.reshape(T, H)
    logits, w0b, w1b, slots, ends8 = _router_meta(hs2, gate_w)
    ends = ends8[0, :E]
    slot0 = slots[:, 0].reshape(NW, NCH_SC, SC_CH)
    slot1 = slots[:, 1].reshape(NW, NCH_SC, SC_CH)
    wc0 = w0b.reshape(NW, NCH_SC, SC_CH * 16)
    wc1 = w1b.reshape(NW, NCH_SC, SC_CH * 16)
    sc_scatter, sc_combine = _sc_kernels()
    x_sorted = sc_scatter(hs2, slot0, slot1)
    y = _grouped_gemm(ends, x_sorted,
                      w1.astype(jnp.bfloat16),
                      w3.astype(jnp.bfloat16),
                      w2.astype(jnp.bfloat16))
    final = sc_combine(y, slot0, slot1, wc0, wc1)
    return final.reshape(b, s, h), logits


# speedup vs baseline: 2.0131x; 1.1628x over previous
import functools

import jax
import jax.numpy as jnp
from jax import lax
from jax.experimental import pallas as pl
from jax.experimental.pallas import tpu as pltpu
from jax.experimental.pallas import tpu_sc as plsc

T, H, F, E, K = 4096, 1024, 2048, 8, 2
BLK = 256
P = T * K + E * BLK
NB = P // BLK
CHUNK = 512
NCH = T // CHUNK

NC_SC, NS_SC = 2, 16
NW = NC_SC * NS_SC
TPW = T // NW
SC_CH = 32
NCH_SC = TPW // SC_CH


def _router_meta_body(hs_ref, gw_ref, logits_ref, w0_ref, w1_ref, slots_ref,
                      ends_ref, carry, eids):
    i = pl.program_id(0)

    @pl.when(i < NCH)
    def _chunk():
        @pl.when(i == 0)
        def _init():
            carry[...] = jnp.zeros_like(carry)

        x = hs_ref[...]
        gw = gw_ref[...]
        logits = lax.dot_general(x, gw, (((1,), (1,)), ((), ())),
                                 preferred_element_type=jnp.float32)
        logits_ref[pl.ds(i * CHUNK, CHUNK), :] = logits

        lane = lax.broadcasted_iota(jnp.int32, (CHUNK, E), 1)
        m0 = jnp.max(logits, axis=1, keepdims=True)
        idx0 = jnp.min(jnp.where(logits == m0, lane, E), axis=1, keepdims=True)
        sel0 = lane == idx0
        l2 = jnp.where(sel0, -jnp.inf, logits)
        m1 = jnp.max(l2, axis=1, keepdims=True)
        idx1 = jnp.min(jnp.where(l2 == m1, lane, E), axis=1, keepdims=True)
        sel1 = lane == idx1

        p = jnp.exp(logits - m0)
        p0 = jnp.sum(jnp.where(sel0, p, 0.0), axis=1, keepdims=True)
        p1 = jnp.sum(jnp.where(sel1, p, 0.0), axis=1, keepdims=True)
        w0 = p0 / (p0 + p1)
        w1 = p1 / (p0 + p1)
        w0_ref[pl.ds(i * CHUNK, CHUNK), :] = jnp.broadcast_to(w0, (CHUNK, 16))
        w1_ref[pl.ds(i * CHUNK, CHUNK), :] = jnp.broadcast_to(w1, (CHUNK, 16))
        eids[pl.ds(i * CHUNK, CHUNK), :] = (
            jnp.where(lane == 0, idx0, 0) + jnp.where(lane == 1, idx1, 0))

        oh0 = sel0.astype(jnp.bfloat16)
        oh1 = sel1.astype(jnp.bfloat16)
        ohs = oh0 + oh1
        row = lax.broadcasted_iota(jnp.int32, (CHUNK, CHUNK), 0)
        col = lax.broadcasted_iota(jnp.int32, (CHUNK, CHUNK), 1)
        tril = (col < row).astype(jnp.bfloat16)
        cum = lax.dot_general(tril, ohs, (((1,), (0,)), ((), ())),
                              preferred_element_type=jnp.float32)
        base = cum + carry[0:1, 0:E]
        r0 = jnp.sum(jnp.where(sel0, base, 0.0), axis=1, keepdims=True)
        r1 = jnp.sum(jnp.where(sel1, base, 0.0), axis=1, keepdims=True)
        slots_ref[pl.ds(i * CHUNK, CHUNK), :] = (
            jnp.where(lane == 0, r0, 0.0)
            + jnp.where(lane == 1, r1, 0.0)).astype(jnp.int32)
        carry[0:1, 0:E] = carry[0:1, 0:E] + jnp.sum(
            ohs.astype(jnp.float32), axis=0, keepdims=True)

    @pl.when(i == NCH)
    def _final():
        counts = carry[0:1, 0:E].astype(jnp.int32)
        padded = ((counts + (BLK - 1)) // BLK) * BLK
        lt = (lax.broadcasted_iota(jnp.int32, (E, E), 0)
              < lax.broadcasted_iota(jnp.int32, (E, E), 1)).astype(jnp.bfloat16)
        offs = lax.dot_general(padded.astype(jnp.bfloat16), lt,
                               (((1,), (0,)), ((), ())),
                               preferred_element_type=jnp.float32)
        offs_i = offs.astype(jnp.int32)
        ends_ref[0:1, 0:E] = offs_i + padded

        lane = lax.broadcasted_iota(jnp.int32, (T, E), 1)
        eall = eids[...]
        e0 = eall[:, 0:1]
        e1 = eall[:, 1:2]
        off0 = jnp.sum(jnp.where(e0 == lane, offs_i, 0), axis=1, keepdims=True)
        off1 = jnp.sum(jnp.where(e1 == lane, offs_i, 0), axis=1, keepdims=True)
        slots_ref[...] = (slots_ref[...]
                          + jnp.where(lane == 0, off0, 0)
                          + jnp.where(lane == 1, off1, 0))


def _router_meta(hs2, gate_w):
    return pl.pallas_call(
        _router_meta_body,
        grid=(NCH + 1,),
        in_specs=[
            pl.BlockSpec((CHUNK, H), lambda i: (jnp.minimum(i, NCH - 1), 0)),
            pl.BlockSpec((E, H), lambda i: (0, 0)),
        ],
        out_specs=[
            pl.BlockSpec((T, E), lambda i: (0, 0)),
            pl.BlockSpec((T, 16), lambda i: (0, 0)),
            pl.BlockSpec((T, 16), lambda i: (0, 0)),
            pl.BlockSpec((T, E), lambda i: (0, 0)),
            pl.BlockSpec((8, 128), lambda i: (0, 0)),
        ],
        out_shape=[
            jax.ShapeDtypeStruct((T, E), jnp.float32),
            jax.ShapeDtypeStruct((T, 16), jnp.float32),
            jax.ShapeDtypeStruct((T, 16), jnp.float32),
            jax.ShapeDtypeStruct((T, E), jnp.int32),
            jax.ShapeDtypeStruct((8, 128), jnp.int32),
        ],
        scratch_shapes=[
            pltpu.VMEM((8, 128), jnp.float32),
            pltpu.VMEM((T, E), jnp.int32),
        ],
    )(hs2, gate_w)


@functools.cache
def _sc_kernels():
    mesh = plsc.VectorSubcoreMesh(core_axis_name="c", subcore_axis_name="s")

    @functools.partial(
        pl.kernel,
        out_type=jax.ShapeDtypeStruct((P, H), jnp.float32),
        mesh=mesh,
        scratch_types=[
            pltpu.VMEM((SC_CH, H), jnp.float32),
            pltpu.VMEM((SC_CH,), jnp.int32),
            pltpu.VMEM((SC_CH,), jnp.int32),
            pltpu.SemaphoreType.DMA,
            pltpu.SemaphoreType.DMA,
        ],
    )
    def sc_scatter(hs_hbm, slot0_hbm, slot1_hbm, x_hbm, xbuf, idx0, idx1,
                   sem0, sem1):
        wid = lax.axis_index("s") * NC_SC + lax.axis_index("c")

        def body(c, _):
            base = wid * TPW + c * SC_CH
            pltpu.sync_copy(hs_hbm.at[pl.ds(base, SC_CH)], xbuf)
            pltpu.sync_copy(slot0_hbm.at[wid, c], idx0)
            pltpu.sync_copy(slot1_hbm.at[wid, c], idx1)
            d0 = pltpu.async_copy(xbuf, x_hbm.at[idx0], sem0)
            d1 = pltpu.async_copy(xbuf, x_hbm.at[idx1], sem1)
            d0.wait()
            d1.wait()
            return ()

        lax.fori_loop(0, NCH_SC, body, ())

    @functools.partial(
        pl.kernel,
        out_type=jax.ShapeDtypeStruct((T, H), jnp.float32),
        mesh=mesh,
        scratch_types=[
            pltpu.VMEM((SC_CH, H), jnp.float32),
            pltpu.VMEM((SC_CH, H), jnp.float32),
            pltpu.VMEM((SC_CH, H), jnp.float32),
            pltpu.VMEM((SC_CH,), jnp.int32),
            pltpu.VMEM((SC_CH,), jnp.int32),
            pltpu.VMEM((SC_CH * 16,), jnp.float32),
            pltpu.VMEM((SC_CH * 16,), jnp.float32),
            pltpu.SemaphoreType.DMA,
            pltpu.SemaphoreType.DMA,
        ],
    )
    def sc_combine(y_hbm, slot0_hbm, slot1_hbm, w0_hbm, w1_hbm, out_hbm,
                   y0, y1, ob, idx0, idx1, wv0, wv1, sem0, sem1):
        wid = lax.axis_index("s") * NC_SC + lax.axis_index("c")

        def body(c, _):
            base = wid * TPW + c * SC_CH
            pltpu.sync_copy(slot0_hbm.at[wid, c], idx0)
            pltpu.sync_copy(slot1_hbm.at[wid, c], idx1)
            pltpu.sync_copy(w0_hbm.at[wid, c], wv0)
            pltpu.sync_copy(w1_hbm.at[wid, c], wv1)
            d0 = pltpu.async_copy(y_hbm.at[idx0], y0, sem0)
            d1 = pltpu.async_copy(y_hbm.at[idx1], y1, sem1)
            d0.wait()
            d1.wait()

            def rows(r, _):
                woff = pl.multiple_of(r * 16, 16)
                wb0 = wv0[pl.ds(woff, 16)]
                wb1 = wv1[pl.ds(woff, 16)]

                def cols(v, _):
                    for u in range(8):
                        off = pl.multiple_of(v * 128 + u * 16, 16)
                        ob[r, pl.ds(off, 16)] = (
                            wb0 * y0[r, pl.ds(off, 16)]
                            + wb1 * y1[r, pl.ds(off, 16)])
                    return ()

                lax.fori_loop(0, H // 128, cols, ())
                return ()

            lax.fori_loop(0, SC_CH, rows, ())
            pltpu.sync_copy(ob, out_hbm.at[pl.ds(base, SC_CH)])
            return ()

        lax.fori_loop(0, NCH_SC, body, ())

    return sc_scatter, sc_combine


def _gemm_body(ends_ref, x_ref, w1_ref, w3_ref, w2_ref, y_ref):
    x = x_ref[...]
    a = lax.dot_general(x, w1_ref[0], (((1,), (1,)), ((), ())),
                        precision=lax.Precision.DEFAULT,
                        preferred_element_type=jnp.float32)
    b = lax.dot_general(x, w3_ref[0], (((1,), (1,)), ((), ())),
                        precision=lax.Precision.DEFAULT,
                        preferred_element_type=jnp.float32)
    g = a * (1.0 / (1.0 + jnp.exp(-a))) * b
    y_ref[...] = lax.dot_general(g, w2_ref[0],
                                 (((1,), (1,)), ((), ())),
                                 precision=lax.Precision.DEFAULT,
                                 preferred_element_type=jnp.float32)


def _block_expert(i, ends_ref):
    s = jnp.int32(0)
    for e in range(E):
        s = s + jnp.where(i * BLK >= ends_ref[e], 1, 0).astype(jnp.int32)
    return jnp.minimum(s, E - 1)


def _grouped_gemm(ends, x_sorted, w1b, w3b, w2b):
    grid_spec = pltpu.PrefetchScalarGridSpec(
        num_scalar_prefetch=1,
        grid=(NB,),
        in_specs=[
            pl.BlockSpec((BLK, H), lambda i, er: (i, 0)),
            pl.BlockSpec((1, F, H), lambda i, er: (_block_expert(i, er), 0, 0)),
            pl.BlockSpec((1, F, H), lambda i, er: (_block_expert(i, er), 0, 0)),
            pl.BlockSpec((1, H, F), lambda i, er: (_block_expert(i, er), 0, 0)),
        ],
        out_specs=pl.BlockSpec((BLK, H), lambda i, er: (i, 0)),
    )
    return pl.pallas_call(
        _gemm_body,
        grid_spec=grid_spec,
        out_shape=jax.ShapeDtypeStruct((P, H), jnp.float32),
    )(ends, x_sorted, w1b, w3b, w2b)


def kernel(hidden_states, gate_w, w1, w3, w2):
    b, s, h = hidden_states.shape
    hs2 = hidden_states.reshape(T, H)
    logits, w0b, w1b, slots, ends8 = _router_meta(hs2, gate_w)
    ends = ends8[0, :E]
    slot0 = slots[:, 0].reshape(NW, NCH_SC, SC_CH)
    slot1 = slots[:, 1].reshape(NW, NCH_SC, SC_CH)
    wc0 = w0b.reshape(NW, NCH_SC, SC_CH * 16)
    wc1 = w1b.reshape(NW, NCH_SC, SC_CH * 16)
    sc_scatter, sc_combine = _sc_kernels()
    x_sorted = sc_scatter(hs2, slot0, slot1)
    y = _grouped_gemm(ends, x_sorted, w1, w3, w2)
    final = sc_combine(y, slot0, slot1, wc0, wc1)
    return final.reshape(b, s, h), logits


# speedup vs baseline: 2.1235x; 1.0548x over previous
import functools

import jax
import jax.numpy as jnp
from jax import lax
from jax.experimental import pallas as pl
from jax.experimental.pallas import tpu as pltpu
from jax.experimental.pallas import tpu_sc as plsc

T, H, F, E, K = 4096, 1024, 2048, 8, 2
BLK = 256
P = T * K + E * BLK
NB = P // BLK
CHUNK = 512
NCH = T // CHUNK

NC_SC, NS_SC = 2, 16
NW = NC_SC * NS_SC
TPW = T // NW
SC_CH = 32
NCH_SC = TPW // SC_CH
SC_CHE = 16
NCHE = TPW // SC_CHE


def _router_meta_body(hs_ref, gw_ref, logits_ref, w0_ref, w1_ref, slots_ref,
                      ends_ref, carry, eids):
    i = pl.program_id(0)

    @pl.when(i < NCH)
    def _chunk():
        @pl.when(i == 0)
        def _init():
            carry[...] = jnp.zeros_like(carry)

        x = hs_ref[...]
        gw = gw_ref[...]
        logits = lax.dot_general(x, gw, (((1,), (1,)), ((), ())),
                                 preferred_element_type=jnp.float32)
        logits_ref[pl.ds(i * CHUNK, CHUNK), :] = logits

        lane = lax.broadcasted_iota(jnp.int32, (CHUNK, E), 1)
        m0 = jnp.max(logits, axis=1, keepdims=True)
        idx0 = jnp.min(jnp.where(logits == m0, lane, E), axis=1, keepdims=True)
        sel0 = lane == idx0
        l2 = jnp.where(sel0, -jnp.inf, logits)
        m1 = jnp.max(l2, axis=1, keepdims=True)
        idx1 = jnp.min(jnp.where(l2 == m1, lane, E), axis=1, keepdims=True)
        sel1 = lane == idx1

        p = jnp.exp(logits - m0)
        p0 = jnp.sum(jnp.where(sel0, p, 0.0), axis=1, keepdims=True)
        p1 = jnp.sum(jnp.where(sel1, p, 0.0), axis=1, keepdims=True)
        w0 = p0 / (p0 + p1)
        w1 = p1 / (p0 + p1)
        w0_ref[pl.ds(i * CHUNK, CHUNK), :] = jnp.broadcast_to(w0, (CHUNK, 16))
        w1_ref[pl.ds(i * CHUNK, CHUNK), :] = jnp.broadcast_to(w1, (CHUNK, 16))
        eids[pl.ds(i * CHUNK, CHUNK), :] = (
            jnp.where(lane == 0, idx0, 0) + jnp.where(lane == 1, idx1, 0))

        oh0 = sel0.astype(jnp.bfloat16)
        oh1 = sel1.astype(jnp.bfloat16)
        ohs = oh0 + oh1
        row = lax.broadcasted_iota(jnp.int32, (CHUNK, CHUNK), 0)
        col = lax.broadcasted_iota(jnp.int32, (CHUNK, CHUNK), 1)
        tril = (col < row).astype(jnp.bfloat16)
        cum = lax.dot_general(tril, ohs, (((1,), (0,)), ((), ())),
                              preferred_element_type=jnp.float32)
        base = cum + carry[0:1, 0:E]
        r0 = jnp.sum(jnp.where(sel0, base, 0.0), axis=1, keepdims=True)
        r1 = jnp.sum(jnp.where(sel1, base, 0.0), axis=1, keepdims=True)
        slots_ref[pl.ds(i * CHUNK, CHUNK), :] = (
            jnp.where(lane == 0, r0, 0.0)
            + jnp.where(lane == 1, r1, 0.0)).astype(jnp.int32)
        carry[0:1, 0:E] = carry[0:1, 0:E] + jnp.sum(
            ohs.astype(jnp.float32), axis=0, keepdims=True)

    @pl.when(i == NCH)
    def _final():
        counts = carry[0:1, 0:E].astype(jnp.int32)
        padded = ((counts + (BLK - 1)) // BLK) * BLK
        lt = (lax.broadcasted_iota(jnp.int32, (E, E), 0)
              < lax.broadcasted_iota(jnp.int32, (E, E), 1)).astype(jnp.bfloat16)
        offs = lax.dot_general(padded.astype(jnp.bfloat16), lt,
                               (((1,), (0,)), ((), ())),
                               preferred_element_type=jnp.float32)
        offs_i = offs.astype(jnp.int32)
        ends_ref[0:1, 0:E] = offs_i + padded

        lane = lax.broadcasted_iota(jnp.int32, (T, E), 1)
        eall = eids[...]
        e0 = eall[:, 0:1]
        e1 = eall[:, 1:2]
        off0 = jnp.sum(jnp.where(e0 == lane, offs_i, 0), axis=1, keepdims=True)
        off1 = jnp.sum(jnp.where(e1 == lane, offs_i, 0), axis=1, keepdims=True)
        slots_ref[...] = (slots_ref[...]
                          + jnp.where(lane == 0, off0, 0)
                          + jnp.where(lane == 1, off1, 0))


def _router_meta(hs2, gate_w):
    return pl.pallas_call(
        _router_meta_body,
        grid=(NCH + 1,),
        in_specs=[
            pl.BlockSpec((CHUNK, H), lambda i: (jnp.minimum(i, NCH - 1), 0)),
            pl.BlockSpec((E, H), lambda i: (0, 0)),
        ],
        out_specs=[
            pl.BlockSpec((T, E), lambda i: (0, 0)),
            pl.BlockSpec((T, 16), lambda i: (0, 0)),
            pl.BlockSpec((T, 16), lambda i: (0, 0)),
            pl.BlockSpec((T, E), lambda i: (0, 0)),
            pl.BlockSpec((8, 128), lambda i: (0, 0)),
        ],
        out_shape=[
            jax.ShapeDtypeStruct((T, E), jnp.float32),
            jax.ShapeDtypeStruct((T, 16), jnp.float32),
            jax.ShapeDtypeStruct((T, 16), jnp.float32),
            jax.ShapeDtypeStruct((T, E), jnp.int32),
            jax.ShapeDtypeStruct((8, 128), jnp.int32),
        ],
        scratch_shapes=[
            pltpu.VMEM((8, 128), jnp.float32),
            pltpu.VMEM((T, E), jnp.int32),
        ],
    )(hs2, gate_w)


@functools.cache
def _sc_kernels():
    mesh = plsc.VectorSubcoreMesh(core_axis_name="c", subcore_axis_name="s")

    @functools.partial(
        pl.kernel,
        out_type=jax.ShapeDtypeStruct((P, H), jnp.float32),
        mesh=mesh,
        scratch_types=[
            pltpu.VMEM((SC_CH, H), jnp.float32),
            pltpu.VMEM((SC_CH, H), jnp.float32),
            pltpu.VMEM((SC_CH,), jnp.int32),
            pltpu.VMEM((SC_CH,), jnp.int32),
            pltpu.VMEM((SC_CH,), jnp.int32),
            pltpu.VMEM((SC_CH,), jnp.int32),
            pltpu.SemaphoreType.DMA,
            pltpu.SemaphoreType.DMA,
            pltpu.SemaphoreType.DMA,
            pltpu.SemaphoreType.DMA,
        ],
    )
    def sc_scatter(hs_hbm, slot0_hbm, slot1_hbm, x_hbm,
                   xbA, xbB, iaA, ibA, iaB, ibB, rsA, rsB, ssA, ssB):
        wid = lax.axis_index("s") * NC_SC + lax.axis_index("c")
        xb = (xbA, xbB)
        ia = (iaA, iaB)
        ib = (ibA, ibB)
        rs = (rsA, rsB)
        ss = (ssA, ssB)

        def start_read(c, b):
            pltpu.sync_copy(slot0_hbm.at[wid, c], ia[b])
            pltpu.sync_copy(slot1_hbm.at[wid, c], ib[b])
            return pltpu.async_copy(
                hs_hbm.at[pl.ds(wid * TPW + c * SC_CH, SC_CH)], xb[b], rs[b])

        rd = {0: start_read(0, 0), 1: start_read(1, 1)}
        sc = {}
        for c in range(NCH_SC):
            b = c % 2
            rd[c].wait()
            sc[c] = (pltpu.async_copy(xb[b], x_hbm.at[ia[b]], ss[b]),
                     pltpu.async_copy(xb[b], x_hbm.at[ib[b]], ss[b]))
            if c + 2 < NCH_SC:
                sc[c][0].wait()
                sc[c][1].wait()
                rd[c + 2] = start_read(c + 2, b)
        for c in range(max(0, NCH_SC - 2), NCH_SC):
            sc[c][0].wait()
            sc[c][1].wait()

    @functools.partial(
        pl.kernel,
        out_type=jax.ShapeDtypeStruct((T, H), jnp.float32),
        mesh=mesh,
        scratch_types=[
            pltpu.VMEM((2 * SC_CHE, H), jnp.float32),
            pltpu.VMEM((2 * SC_CHE, H), jnp.float32),
            pltpu.VMEM((SC_CHE, H), jnp.float32),
            pltpu.VMEM((SC_CHE, H), jnp.float32),
            pltpu.VMEM((2 * SC_CHE,), jnp.int32),
            pltpu.VMEM((2 * SC_CHE,), jnp.int32),
            pltpu.VMEM((TPW * 16,), jnp.float32),
            pltpu.VMEM((TPW * 16,), jnp.float32),
            pltpu.SemaphoreType.DMA,
            pltpu.SemaphoreType.DMA,
            pltpu.SemaphoreType.DMA,
            pltpu.SemaphoreType.DMA,
        ],
    )
    def sc_combine(y_hbm, scidx_hbm, w0_hbm, w1_hbm, out_hbm,
                   ybA, ybB, obA, obB, giA, giB, wv0, wv1,
                   gsA, gsB, osA, osB):
        wid = lax.axis_index("s") * NC_SC + lax.axis_index("c")
        yb = (ybA, ybB)
        ob = (obA, obB)
        gi = (giA, giB)
        gs = (gsA, gsB)
        os_ = (osA, osB)
        pltpu.sync_copy(w0_hbm.at[wid], wv0)
        pltpu.sync_copy(w1_hbm.at[wid], wv1)

        def start_gather(c, b):
            pltpu.sync_copy(scidx_hbm.at[wid, c], gi[b])
            return pltpu.async_copy(y_hbm.at[gi[b]], yb[b], gs[b])

        gd = {0: start_gather(0, 0), 1: start_gather(1, 1)}
        wr = {}
        for c in range(NCHE):
            b = c % 2
            gd[c].wait()
            if c >= 2:
                wr[c - 2].wait()
            ybc = yb[b]
            obc = ob[b]

            def rows(r, _):
                woff = pl.multiple_of(c * SC_CHE * 16 + r * 16, 16)
                wb0 = wv0[pl.ds(woff, 16)]
                wb1 = wv1[pl.ds(woff, 16)]

                def cols(v, _):
                    for u in range(8):
                        off = pl.multiple_of(v * 128 + u * 16, 16)
                        obc[r, pl.ds(off, 16)] = (
                            wb0 * ybc[r, pl.ds(off, 16)]
                            + wb1 * ybc[r + SC_CHE, pl.ds(off, 16)])
                    return ()

                lax.fori_loop(0, H // 128, cols, ())
                return ()

            lax.fori_loop(0, SC_CHE, rows, ())
            wr[c] = pltpu.async_copy(
                ob[b], out_hbm.at[pl.ds(wid * TPW + c * SC_CHE, SC_CHE)],
                os_[b])
            if c + 2 < NCHE:
                gd[c + 2] = start_gather(c + 2, b)
        for c in range(max(0, NCHE - 2), NCHE):
            wr[c].wait()

    return sc_scatter, sc_combine


def _gemm_body(ends_ref, x_ref, w1_ref, w3_ref, w2_ref, y_ref):
    x = x_ref[...]
    a = lax.dot_general(x, w1_ref[0], (((1,), (1,)), ((), ())),
                        precision=lax.Precision.DEFAULT,
                        preferred_element_type=jnp.float32)
    b = lax.dot_general(x, w3_ref[0], (((1,), (1,)), ((), ())),
                        precision=lax.Precision.DEFAULT,
                        preferred_element_type=jnp.float32)
    g = a * (1.0 / (1.0 + jnp.exp(-a))) * b
    y_ref[...] = lax.dot_general(g, w2_ref[0],
                                 (((1,), (1,)), ((), ())),
                                 precision=lax.Precision.DEFAULT,
                                 preferred_element_type=jnp.float32)


def _block_expert(i, ends_ref):
    s = jnp.int32(0)
    for e in range(E):
        s = s + jnp.where(i * BLK >= ends_ref[e], 1, 0).astype(jnp.int32)
    return jnp.minimum(s, E - 1)


def _grouped_gemm(ends, x_sorted, w1b, w3b, w2b):
    grid_spec = pltpu.PrefetchScalarGridSpec(
        num_scalar_prefetch=1,
        grid=(NB,),
        in_specs=[
            pl.BlockSpec((BLK, H), lambda i, er: (i, 0)),
            pl.BlockSpec((1, F, H), lambda i, er: (_block_expert(i, er), 0, 0)),
            pl.BlockSpec((1, F, H), lambda i, er: (_block_expert(i, er), 0, 0)),
            pl.BlockSpec((1, H, F), lambda i, er: (_block_expert(i, er), 0, 0)),
        ],
        out_specs=pl.BlockSpec((BLK, H), lambda i, er: (i, 0)),
    )
    return pl.pallas_call(
        _gemm_body,
        grid_spec=grid_spec,
        out_shape=jax.ShapeDtypeStruct((P, H), jnp.float32),
    )(ends, x_sorted, w1b, w3b, w2b)


def kernel(hidden_states, gate_w, w1, w3, w2):
    b, s, h = hidden_states.shape
    hs2 = hidden_states.reshape(T, H)
    logits, w0b, w1b, slots, ends8 = _router_meta(hs2, gate_w)
    ends = ends8[0, :E]
    slot0 = slots[:, 0].reshape(NW, NCH_SC, SC_CH)
    slot1 = slots[:, 1].reshape(NW, NCH_SC, SC_CH)
    scidx = jnp.concatenate(
        [slots[:, 0].reshape(NW, NCHE, SC_CHE),
         slots[:, 1].reshape(NW, NCHE, SC_CHE)], axis=-1)
    wflat0 = w0b.reshape(NW, TPW * 16)
    wflat1 = w1b.reshape(NW, TPW * 16)
    sc_scatter, sc_combine = _sc_kernels()
    x_sorted = sc_scatter(hs2, slot0, slot1)
    y = _grouped_gemm(ends, x_sorted, w1, w3, w2)
    final = sc_combine(y, scidx, wflat0, wflat1)
    return final.reshape(b, s, h), logits


# speedup vs baseline: 2.3125x; 1.0890x over previous
import functools

import jax
import jax.numpy as jnp
from jax import lax
from jax.experimental import pallas as pl
from jax.experimental.pallas import tpu as pltpu
from jax.experimental.pallas import tpu_sc as plsc

T, H, F, E, K = 4096, 1024, 2048, 8, 2
BLK = 512
P = T * K + E * BLK
NB = P // BLK
CHUNK = 512
NCH = T // CHUNK

NC_SC, NS_SC = 2, 16
NW = NC_SC * NS_SC
TPW = T // NW
SC_CH = 32
NCH_SC = TPW // SC_CH
SC_CHE = 16
NCHE = TPW // SC_CHE


def _router_meta_body(hs_ref, gw_ref, logits_ref, w0_ref, w1_ref, slots_ref,
                      ends_ref, carry, eids):
    i = pl.program_id(0)

    @pl.when(i < NCH)
    def _chunk():
        @pl.when(i == 0)
        def _init():
            carry[...] = jnp.zeros_like(carry)

        x = hs_ref[...]
        gw = gw_ref[...]
        logits = lax.dot_general(x, gw, (((1,), (1,)), ((), ())),
                                 preferred_element_type=jnp.float32)
        logits_ref[pl.ds(i * CHUNK, CHUNK), :] = logits

        lane = lax.broadcasted_iota(jnp.int32, (CHUNK, E), 1)
        m0 = jnp.max(logits, axis=1, keepdims=True)
        idx0 = jnp.min(jnp.where(logits == m0, lane, E), axis=1, keepdims=True)
        sel0 = lane == idx0
        l2 = jnp.where(sel0, -jnp.inf, logits)
        m1 = jnp.max(l2, axis=1, keepdims=True)
        idx1 = jnp.min(jnp.where(l2 == m1, lane, E), axis=1, keepdims=True)
        sel1 = lane == idx1

        p = jnp.exp(logits - m0)
        p0 = jnp.sum(jnp.where(sel0, p, 0.0), axis=1, keepdims=True)
        p1 = jnp.sum(jnp.where(sel1, p, 0.0), axis=1, keepdims=True)
        w0 = p0 / (p0 + p1)
        w1 = p1 / (p0 + p1)
        w0_ref[pl.ds(i * CHUNK, CHUNK), :] = jnp.broadcast_to(w0, (CHUNK, 16))
        w1_ref[pl.ds(i * CHUNK, CHUNK), :] = jnp.broadcast_to(w1, (CHUNK, 16))
        eids[pl.ds(i * CHUNK, CHUNK), :] = (
            jnp.where(lane == 0, idx0, 0) + jnp.where(lane == 1, idx1, 0))

        oh0 = sel0.astype(jnp.bfloat16)
        oh1 = sel1.astype(jnp.bfloat16)
        ohs = oh0 + oh1
        row = lax.broadcasted_iota(jnp.int32, (CHUNK, CHUNK), 0)
        col = lax.broadcasted_iota(jnp.int32, (CHUNK, CHUNK), 1)
        tril = (col < row).astype(jnp.bfloat16)
        cum = lax.dot_general(tril, ohs, (((1,), (0,)), ((), ())),
                              preferred_element_type=jnp.float32)
        base = cum + carry[0:1, 0:E]
        r0 = jnp.sum(jnp.where(sel0, base, 0.0), axis=1, keepdims=True)
        r1 = jnp.sum(jnp.where(sel1, base, 0.0), axis=1, keepdims=True)
        slots_ref[pl.ds(i * CHUNK, CHUNK), :] = (
            jnp.where(lane == 0, r0, 0.0)
            + jnp.where(lane == 1, r1, 0.0)).astype(jnp.int32)
        carry[0:1, 0:E] = carry[0:1, 0:E] + jnp.sum(
            ohs.astype(jnp.float32), axis=0, keepdims=True)

    @pl.when(i == NCH)
    def _final():
        counts = carry[0:1, 0:E].astype(jnp.int32)
        padded = ((counts + (BLK - 1)) // BLK) * BLK
        lt = (lax.broadcasted_iota(jnp.int32, (E, E), 0)
              < lax.broadcasted_iota(jnp.int32, (E, E), 1)).astype(jnp.bfloat16)
        offs = lax.dot_general(padded.astype(jnp.bfloat16), lt,
                               (((1,), (0,)), ((), ())),
                               preferred_element_type=jnp.float32)
        offs_i = offs.astype(jnp.int32)
        ends_ref[0:1, 0:E] = offs_i + padded

        lane = lax.broadcasted_iota(jnp.int32, (T, E), 1)
        eall = eids[...]
        e0 = eall[:, 0:1]
        e1 = eall[:, 1:2]
        off0 = jnp.sum(jnp.where(e0 == lane, offs_i, 0), axis=1, keepdims=True)
        off1 = jnp.sum(jnp.where(e1 == lane, offs_i, 0), axis=1, keepdims=True)
        slots_ref[...] = (slots_ref[...]
                          + jnp.where(lane == 0, off0, 0)
                          + jnp.where(lane == 1, off1, 0))


def _router_meta(hs2, gate_w):
    return pl.pallas_call(
        _router_meta_body,
        grid=(NCH + 1,),
        in_specs=[
            pl.BlockSpec((CHUNK, H), lambda i: (jnp.minimum(i, NCH - 1), 0)),
            pl.BlockSpec((E, H), lambda i: (0, 0)),
        ],
        out_specs=[
            pl.BlockSpec((T, E), lambda i: (0, 0)),
            pl.BlockSpec((T, 16), lambda i: (0, 0)),
            pl.BlockSpec((T, 16), lambda i: (0, 0)),
            pl.BlockSpec((T, E), lambda i: (0, 0)),
            pl.BlockSpec((8, 128), lambda i: (0, 0)),
        ],
        out_shape=[
            jax.ShapeDtypeStruct((T, E), jnp.float32),
            jax.ShapeDtypeStruct((T, 16), jnp.float32),
            jax.ShapeDtypeStruct((T, 16), jnp.float32),
            jax.ShapeDtypeStruct((T, E), jnp.int32),
            jax.ShapeDtypeStruct((8, 128), jnp.int32),
        ],
        scratch_shapes=[
            pltpu.VMEM((8, 128), jnp.float32),
            pltpu.VMEM((T, E), jnp.int32),
        ],
    )(hs2, gate_w)


@functools.cache
def _sc_kernels():
    mesh = plsc.VectorSubcoreMesh(core_axis_name="c", subcore_axis_name="s")

    @functools.partial(
        pl.kernel,
        out_type=jax.ShapeDtypeStruct((P, H), jnp.float32),
        mesh=mesh,
        scratch_types=[
            pltpu.VMEM((SC_CH, H), jnp.float32),
            pltpu.VMEM((SC_CH, H), jnp.float32),
            pltpu.VMEM((SC_CH,), jnp.int32),
            pltpu.VMEM((SC_CH,), jnp.int32),
            pltpu.VMEM((SC_CH,), jnp.int32),
            pltpu.VMEM((SC_CH,), jnp.int32),
            pltpu.SemaphoreType.DMA,
            pltpu.SemaphoreType.DMA,
            pltpu.SemaphoreType.DMA,
            pltpu.SemaphoreType.DMA,
        ],
    )
    def sc_scatter(hs_hbm, slot0_hbm, slot1_hbm, x_hbm,
                   xbA, xbB, iaA, ibA, iaB, ibB, rsA, rsB, ssA, ssB):
        wid = lax.axis_index("s") * NC_SC + lax.axis_index("c")
        xb = (xbA, xbB)
        ia = (iaA, iaB)
        ib = (ibA, ibB)
        rs = (rsA, rsB)
        ss = (ssA, ssB)

        def start_read(c, b):
            pltpu.sync_copy(slot0_hbm.at[wid, c], ia[b])
            pltpu.sync_copy(slot1_hbm.at[wid, c], ib[b])
            return pltpu.async_copy(
                hs_hbm.at[pl.ds(wid * TPW + c * SC_CH, SC_CH)], xb[b], rs[b])

        rd = {0: start_read(0, 0), 1: start_read(1, 1)}
        sc = {}
        for c in range(NCH_SC):
            b = c % 2
            rd[c].wait()
            sc[c] = (pltpu.async_copy(xb[b], x_hbm.at[ia[b]], ss[b]),
                     pltpu.async_copy(xb[b], x_hbm.at[ib[b]], ss[b]))
            if c + 2 < NCH_SC:
                sc[c][0].wait()
                sc[c][1].wait()
                rd[c + 2] = start_read(c + 2, b)
        for c in range(max(0, NCH_SC - 2), NCH_SC):
            sc[c][0].wait()
            sc[c][1].wait()

    @functools.partial(
        pl.kernel,
        out_type=jax.ShapeDtypeStruct((T, H), jnp.float32),
        mesh=mesh,
        scratch_types=[
            pltpu.VMEM((2 * SC_CHE, H), jnp.float32),
            pltpu.VMEM((2 * SC_CHE, H), jnp.float32),
            pltpu.VMEM((SC_CHE, H), jnp.float32),
            pltpu.VMEM((SC_CHE, H), jnp.float32),
            pltpu.VMEM((2 * SC_CHE,), jnp.int32),
            pltpu.VMEM((2 * SC_CHE,), jnp.int32),
            pltpu.VMEM((TPW * 16,), jnp.float32),
            pltpu.VMEM((TPW * 16,), jnp.float32),
            pltpu.SemaphoreType.DMA,
            pltpu.SemaphoreType.DMA,
            pltpu.SemaphoreType.DMA,
            pltpu.SemaphoreType.DMA,
        ],
    )
    def sc_combine(y_hbm, scidx_hbm, w0_hbm, w1_hbm, out_hbm,
                   ybA, ybB, obA, obB, giA, giB, wv0, wv1,
                   gsA, gsB, osA, osB):
        wid = lax.axis_index("s") * NC_SC + lax.axis_index("c")
        yb = (ybA, ybB)
        ob = (obA, obB)
        gi = (giA, giB)
        gs = (gsA, gsB)
        os_ = (osA, osB)
        pltpu.sync_copy(w0_hbm.at[wid], wv0)
        pltpu.sync_copy(w1_hbm.at[wid], wv1)

        def start_gather(c, b):
            pltpu.sync_copy(scidx_hbm.at[wid, c], gi[b])
            return pltpu.async_copy(y_hbm.at[gi[b]], yb[b], gs[b])

        gd = {0: start_gather(0, 0), 1: start_gather(1, 1)}
        wr = {}
        for c in range(NCHE):
            b = c % 2
            gd[c].wait()
            if c >= 2:
                wr[c - 2].wait()
            ybc = yb[b]
            obc = ob[b]

            def rows(r, _):
                woff = pl.multiple_of(c * SC_CHE * 16 + r * 16, 16)
                wb0 = wv0[pl.ds(woff, 16)]
                wb1 = wv1[pl.ds(woff, 16)]

                def cols(v, _):
                    for u in range(8):
                        off = pl.multiple_of(v * 128 + u * 16, 16)
                        obc[r, pl.ds(off, 16)] = (
                            wb0 * ybc[r, pl.ds(off, 16)]
                            + wb1 * ybc[r + SC_CHE, pl.ds(off, 16)])
                    return ()

                lax.fori_loop(0, H // 128, cols, ())
                return ()

            lax.fori_loop(0, SC_CHE, rows, ())
            wr[c] = pltpu.async_copy(
                ob[b], out_hbm.at[pl.ds(wid * TPW + c * SC_CHE, SC_CHE)],
                os_[b])
            if c + 2 < NCHE:
                gd[c + 2] = start_gather(c + 2, b)
        for c in range(max(0, NCHE - 2), NCHE):
            wr[c].wait()

    return sc_scatter, sc_combine


def _gemm_body(ends_ref, x_ref, w1_ref, w3_ref, w2_ref, y_ref):
    x = x_ref[...]
    FH = F // 2
    y = jnp.zeros((BLK, H), jnp.float32)
    for f in range(2):
        w1h = w1_ref[0, pl.ds(f * FH, FH), :]
        w3h = w3_ref[0, pl.ds(f * FH, FH), :]
        w2h = w2_ref[0, :, pl.ds(f * FH, FH)]
        a = lax.dot_general(x, w1h, (((1,), (1,)), ((), ())),
                            precision=lax.Precision.DEFAULT,
                            preferred_element_type=jnp.float32)
        b = lax.dot_general(x, w3h, (((1,), (1,)), ((), ())),
                            precision=lax.Precision.DEFAULT,
                            preferred_element_type=jnp.float32)
        g = a * (1.0 / (1.0 + jnp.exp(-a))) * b
        y = y + lax.dot_general(g, w2h, (((1,), (1,)), ((), ())),
                                precision=lax.Precision.DEFAULT,
                                preferred_element_type=jnp.float32)
    y_ref[...] = y


def _block_expert(i, ends_ref):
    s = jnp.int32(0)
    for e in range(E):
        s = s + jnp.where(i * BLK >= ends_ref[e], 1, 0).astype(jnp.int32)
    return jnp.minimum(s, E - 1)


def _grouped_gemm(ends, x_sorted, w1b, w3b, w2b):
    grid_spec = pltpu.PrefetchScalarGridSpec(
        num_scalar_prefetch=1,
        grid=(NB,),
        in_specs=[
            pl.BlockSpec((BLK, H), lambda i, er: (i, 0)),
            pl.BlockSpec((1, F, H), lambda i, er: (_block_expert(i, er), 0, 0)),
            pl.BlockSpec((1, F, H), lambda i, er: (_block_expert(i, er), 0, 0)),
            pl.BlockSpec((1, H, F), lambda i, er: (_block_expert(i, er), 0, 0)),
        ],
        out_specs=pl.BlockSpec((BLK, H), lambda i, er: (i, 0)),
    )
    return pl.pallas_call(
        _gemm_body,
        grid_spec=grid_spec,
        out_shape=jax.ShapeDtypeStruct((P, H), jnp.float32),
        compiler_params=pltpu.CompilerParams(
            vmem_limit_bytes=64 * 1024 * 1024),
    )(ends, x_sorted, w1b, w3b, w2b)


def kernel(hidden_states, gate_w, w1, w3, w2):
    b, s, h = hidden_states.shape
    hs2 = hidden_states.reshape(T, H)
    logits, w0b, w1b, slots, ends8 = _router_meta(hs2, gate_w)
    ends = ends8[0, :E]
    slot0 = slots[:, 0].reshape(NW, NCH_SC, SC_CH)
    slot1 = slots[:, 1].reshape(NW, NCH_SC, SC_CH)
    scidx = jnp.concatenate(
        [slots[:, 0].reshape(NW, NCHE, SC_CHE),
         slots[:, 1].reshape(NW, NCHE, SC_CHE)], axis=-1)
    wflat0 = w0b.reshape(NW, TPW * 16)
    wflat1 = w1b.reshape(NW, TPW * 16)
    sc_scatter, sc_combine = _sc_kernels()
    x_sorted = sc_scatter(hs2, slot0, slot1)
    y = _grouped_gemm(ends, x_sorted, w1, w3, w2)
    final = sc_combine(y, scidx, wflat0, wflat1)
    return final.reshape(b, s, h), logits
